# baseline (speedup 1.0000x reference)
import functools

import numpy as np
import jax
import jax.numpy as jnp
from jax import lax
from jax.experimental import pallas as pl
from jax.experimental.pallas import tpu as pltpu
from jax.experimental.pallas import tpu_sc as plsc

_D_IN = 16
_N = 26
_SIZES = [_N - i for i in range(_N)]
_TOTAL = sum(_SIZES)
_OFFS = [int(v) for v in np.cumsum([0] + _SIZES)]

_BB = 2048
_NC = 2
_NS = 16
_NW = _NC * _NS


def _matmul_body(x_ref, w_ref, b_ref, y_ref):
    y_ref[...] = jax.lax.dot_general(
        x_ref[...], w_ref[...], (((1,), (1,)), ((), ())),
        preferred_element_type=jnp.float32) + b_ref[...]


def _tc_logits(x, W_cat, b_cat):
    B = x.shape[0]
    return pl.pallas_call(
        _matmul_body,
        grid=(B // _BB,),
        in_specs=[
            pl.BlockSpec((_BB, _D_IN), lambda i: (i, 0)),
            pl.BlockSpec((_TOTAL, _D_IN), lambda i: (0, 0)),
            pl.BlockSpec((1, _TOTAL), lambda i: (0, 0)),
        ],
        out_specs=pl.BlockSpec((_BB, _TOTAL), lambda i: (i, 0)),
        out_shape=jax.ShapeDtypeStruct((B, _TOTAL), jnp.float32),
    )(x, W_cat, b_cat[None, :])


_RB = 128
_L = 16

_WINDOWS = []
for _i in range(_N):
    _k = _SIZES[_i]
    _off = _OFFS[_i]
    _c = 0
    while _c < _k:
        _n = min(_L, _k - _c)
        _ws = min(_off + _c, _TOTAL - _L)
        _ph = _off + _c - _ws
        _WINDOWS.append((_i, _ws, _ph, _c, _n))
        _c += _n


def _repack_body(y_hbm, *refs):
    outs = refs[:_N]
    obufs = refs[_N:2 * _N]
    ybuf = refs[2 * _N]
    rows_w = y_hbm.shape[0] // _NW
    wid = lax.axis_index("s") * _NC + lax.axis_index("c")
    r0 = wid * rows_w

    iota = lax.iota(jnp.int32, _L)

    def row_body(r, carry):
        rvec = jnp.full((_L,), r, jnp.int32)
        for (i, ws, ph, c, n) in _WINDOWS:
            vals = ybuf[r, pl.ds(ws, _L)]
            cols = iota - ph + c
            mask = (iota >= ph) & (iota < ph + n)
            plsc.store_scatter(obufs[i], [rvec, cols], vals, mask=mask)
        return carry

    for blk in range(rows_w // _RB):
        rb = r0 + blk * _RB
        pltpu.sync_copy(y_hbm.at[pl.ds(rb, _RB), :], ybuf)
        lax.fori_loop(0, _RB, row_body, 0, unroll=4)
        for i in range(_N):
            pltpu.sync_copy(obufs[i], outs[i].at[pl.ds(rb, _RB), :])


def _sc_repack(y):
    B = y.shape[0]
    rows = B // _NW
    mesh = plsc.VectorSubcoreMesh(core_axis_name="c", subcore_axis_name="s")
    fn = pl.kernel(
        _repack_body,
        mesh=mesh,
        out_type=[
            jax.ShapeDtypeStruct((B, _SIZES[i]), jnp.float32)
            for i in range(_N)
        ],
        scratch_types=(
            [pltpu.VMEM((_RB, _SIZES[i]), jnp.float32) for i in range(_N)]
            + [pltpu.VMEM((_RB, _TOTAL), jnp.float32)]
        ),
        compiler_params=pltpu.CompilerParams(
            use_tc_tiling_on_sc=False, needs_layout_passes=False),
    )
    return fn(y)


def kernel(x, W_cat, b_cat):
    y = _tc_logits(x, W_cat, b_cat)
    return tuple(_sc_repack(y))


# speedup vs baseline: 1.3423x; 1.3423x over previous
import functools

import numpy as np
import jax
import jax.numpy as jnp
from jax import lax
from jax.experimental import pallas as pl
from jax.experimental.pallas import tpu as pltpu
from jax.experimental.pallas import tpu_sc as plsc

_D_IN = 16
_N = 26
_SIZES = [_N - i for i in range(_N)]
_TOTAL = sum(_SIZES)
_OFFS = [int(v) for v in np.cumsum([0] + _SIZES)]

_BB = 2048
_NC = 2
_NS = 16
_NW = _NC * _NS


def _matmul_body(x_ref, w_ref, b_ref, y_ref):
    y_ref[...] = jax.lax.dot_general(
        x_ref[...], w_ref[...], (((1,), (1,)), ((), ())),
        preferred_element_type=jnp.float32) + b_ref[...]


def _tc_logits(x, W_cat, b_cat):
    B = x.shape[0]
    return pl.pallas_call(
        _matmul_body,
        grid=(B // _BB,),
        in_specs=[
            pl.BlockSpec((_BB, _D_IN), lambda i: (i, 0)),
            pl.BlockSpec((_TOTAL, _D_IN), lambda i: (0, 0)),
            pl.BlockSpec((1, _TOTAL), lambda i: (0, 0)),
        ],
        out_specs=pl.BlockSpec((_BB, _TOTAL), lambda i: (i, 0)),
        out_shape=jax.ShapeDtypeStruct((B, _TOTAL), jnp.float32),
    )(x, W_cat, b_cat[None, :])


_RB = 128
_L = 16

_WINDOWS = []
for _i in range(_N):
    _k = _SIZES[_i]
    _off = _OFFS[_i]
    _c = 0
    while _c < _k:
        _n = min(_L, _k - _c)
        _ws = min(_off + _c, _TOTAL - _L)
        _ph = _off + _c - _ws
        _WINDOWS.append((_i, _ws, _ph, _c, _n))
        _c += _n


def _repack_body(y_hbm, *refs):
    outs = refs[:_N]
    obufs = refs[_N:2 * _N]
    ybuf = refs[2 * _N]
    rows_w = y_hbm.shape[0] // _NW
    wid = lax.axis_index("s") * _NC + lax.axis_index("c")
    r0 = wid * rows_w

    iota = lax.iota(jnp.int32, _L)

    def row_body(r, carry):
        rvec = jnp.full((_L,), r, jnp.int32)
        for (i, ws, ph, c, n) in _WINDOWS:
            vals = ybuf[r, pl.ds(ws, _L)]
            cols = iota - ph + c
            mask = (iota >= ph) & (iota < ph + n)
            plsc.store_scatter(obufs[i], [rvec, cols], vals, mask=mask)
        return carry

    for blk in range(rows_w // _RB):
        rb = r0 + blk * _RB
        pltpu.sync_copy(y_hbm.at[pl.ds(rb, _RB), :], ybuf)
        lax.fori_loop(0, _RB, row_body, 0, unroll=4)
        for i in range(_N):
            pltpu.sync_copy(obufs[i], outs[i].at[pl.ds(rb, _RB), :])


def _sc_repack(y):
    B = y.shape[0]
    rows = B // _NW
    mesh = plsc.VectorSubcoreMesh(core_axis_name="c", subcore_axis_name="s")
    fn = pl.kernel(
        _repack_body,
        mesh=mesh,
        out_type=[
            jax.ShapeDtypeStruct((B, _SIZES[i]), jnp.float32)
            for i in range(_N)
        ],
        scratch_types=(
            [pltpu.VMEM((_RB, _SIZES[i]), jnp.float32) for i in range(_N)]
            + [pltpu.VMEM((_RB, _TOTAL), jnp.float32)]
        ),
        compiler_params=pltpu.CompilerParams(
            use_tc_tiling_on_sc=False, needs_layout_passes=False),
    )
    return fn(y)


def kernel(x, W_cat, b_cat):
    y = _tc_logits(x, W_cat, b_cat)
    return tuple(y[:, _OFFS[i]:_OFFS[i + 1]] for i in range(_N))
